# baseline (speedup 1.0000x reference)
import functools

import jax
import jax.numpy as jnp
from jax import lax
from jax.experimental import pallas as pl
from jax.experimental.pallas import tpu as pltpu
from jax.experimental.pallas import tpu_sc as plsc

NC, NS, L = 2, 16, 16
NW = NC * NS

VOCAB = 1000000
CHUNK = 16
N_CHUNKS = 4
BATCH = 16384
ROWS = 65536

B_PER_W = BATCH // NW
IDX_CHUNK = 128
N_XCH = B_PER_W // IDX_CHUNK
FLAT_PER_W = B_PER_W * N_CHUNKS
N_TCH = FLAT_PER_W // IDX_CHUNK
OUT_ROWS = BATCH * N_CHUNKS * CHUNK // 128
OROWS_PER_W = OUT_ROWS // NW

_mesh = plsc.VectorSubcoreMesh(
    core_axis_name="c", subcore_axis_name="s", num_cores=NC, num_subcores=NS)


@functools.partial(
    pl.kernel,
    out_type=jax.ShapeDtypeStruct((OUT_ROWS, 128), jnp.float32),
    mesh=_mesh,
    scratch_types=[
        pltpu.VMEM((B_PER_W,), jnp.int32),
        pltpu.VMEM((N_XCH, 4 * IDX_CHUNK), jnp.int32),
        pltpu.VMEM((2 * B_PER_W, N_CHUNKS), jnp.int32),
        pltpu.VMEM((2 * B_PER_W, N_CHUNKS), jnp.int32),
        pltpu.VMEM((N_TCH, IDX_CHUNK), jnp.int32),
        pltpu.VMEM((N_TCH, IDX_CHUNK), jnp.int32),
        pltpu.VMEM((2 * IDX_CHUNK, N_CHUNKS, CHUNK), jnp.float32),
        pltpu.VMEM((2 * IDX_CHUNK, N_CHUNKS, CHUNK), jnp.float32),
        pltpu.VMEM((OROWS_PER_W, 128), jnp.float32),
        pltpu.SemaphoreType.DMA,
        pltpu.SemaphoreType.DMA,
    ],
    compiler_params=pltpu.CompilerParams(
        use_tc_tiling_on_sc=False, needs_layout_passes=False),
)
def _cc_embed(x_hbm, t0_hbm, t1_hbm, h0_hbm, h1_hbm, out_hbm,
              xv, xe, r0, r1, f0, f1, gb0, gb1, gout, sem0, sem1):
    wid = lax.axis_index("s") * NC + lax.axis_index("c")
    lane = lax.iota(jnp.int32, L)
    zeros = lane * 0
    col = lane & (N_CHUNKS - 1)

    pltpu.sync_copy(x_hbm.at[pl.ds(wid * B_PER_W, B_PER_W)], xv)
    for j in range(N_XCH):
        rowj = jnp.broadcast_to(jnp.int32(j), (L,))
        for m in range(IDX_CHUNK // L):
            v = xv[pl.ds(j * IDX_CHUNK + m * L, L)]
            ce = (m * L + lane) * 2
            plsc.store_scatter(xe, [rowj, ce], v + v)
            plsc.store_scatter(xe, [rowj, ce + 1], zeros)
        for m in range(2 * IDX_CHUNK // L, 4 * IDX_CHUNK // L):
            xe[j, pl.ds(m * L, L)] = zeros
    pl.delay(128)
    cps = []
    for j in range(N_XCH):
        cps.append(pltpu.async_copy(
            h0_hbm.at[xe.at[j, pl.ds(0, 2 * IDX_CHUNK)]],
            r0.at[pl.ds(j * 2 * IDX_CHUNK, 2 * IDX_CHUNK)], sem0))
        cps.append(pltpu.async_copy(
            h1_hbm.at[xe.at[j, pl.ds(0, 2 * IDX_CHUNK)]],
            r1.at[pl.ds(j * 2 * IDX_CHUNK, 2 * IDX_CHUNK)], sem1))
    for c in cps:
        c.wait()

    for j in range(N_TCH):
        for k in range(IDX_CHUNK // L):
            base = j * IDX_CHUNK + k * L
            bvec = lax.shift_right_logical(base + lane, 2)
            rowi = bvec + (bvec & -IDX_CHUNK)
            f0[j, pl.ds(k * L, L)] = plsc.load_gather(r0, [rowi, col])
            f1[j, pl.ds(k * L, L)] = plsc.load_gather(r1, [rowi, col])

    pl.delay(128)

    def _start(j):
        buf = (j & 1) * IDX_CHUNK
        a = pltpu.async_copy(
            t0_hbm.at[f0.at[j]], gb0.at[pl.ds(buf, IDX_CHUNK)], sem0)
        b = pltpu.async_copy(
            t1_hbm.at[f1.at[j]], gb1.at[pl.ds(buf, IDX_CHUNK)], sem1)
        return a, b

    pend = _start(0)
    for j in range(N_TCH):
        pend[0].wait()
        pend[1].wait()
        if j + 1 < N_TCH:
            pend = _start(j + 1)
        bufbase = (j & 1) * IDX_CHUNK

        @pl.loop(0, IDX_CHUNK, unroll=4)
        def _extract(i):
            rowv = jnp.broadcast_to(bufbase + i, (L,))
            cv = jnp.broadcast_to(i & (N_CHUNKS - 1), (L,))
            v = (plsc.load_gather(gb0, [rowv, cv, lane])
                 + plsc.load_gather(gb1, [rowv, cv, lane]))
            p = j * IDX_CHUNK + i
            gout[lax.shift_right_logical(p, 3),
                 pl.ds((i & 7) * L, L)] = v

    pltpu.sync_copy(gout, out_hbm.at[pl.ds(wid * OROWS_PER_W, OROWS_PER_W)])


def kernel(x, table0, table1, h0, h1):
    out = _cc_embed(x, table0, table1, h0, h1)
    return out.reshape(BATCH, N_CHUNKS * CHUNK)


# speedup vs baseline: 1.2517x; 1.2517x over previous
import functools

import jax
import jax.numpy as jnp
from jax import lax
from jax.experimental import pallas as pl
from jax.experimental.pallas import tpu as pltpu
from jax.experimental.pallas import tpu_sc as plsc

NC, NS, L = 2, 16, 16
NW = NC * NS

VOCAB = 1000000
CHUNK = 16
N_CHUNKS = 4
BATCH = 16384
ROWS = 65536

B_PER_W = BATCH // NW
IDX_CHUNK = 128
N_XCH = B_PER_W // IDX_CHUNK
FLAT_PER_W = B_PER_W * N_CHUNKS
N_TCH = FLAT_PER_W // IDX_CHUNK

_mesh = plsc.VectorSubcoreMesh(
    core_axis_name="c", subcore_axis_name="s", num_cores=NC, num_subcores=NS)


@functools.partial(
    pl.kernel,
    out_type=jax.ShapeDtypeStruct((BATCH * N_CHUNKS, CHUNK), jnp.float32),
    mesh=_mesh,
    scratch_types=[
        pltpu.VMEM((B_PER_W,), jnp.int32),
        pltpu.VMEM((N_XCH, IDX_CHUNK), jnp.int32),
        pltpu.VMEM((B_PER_W, CHUNK), jnp.int32),
        pltpu.VMEM((B_PER_W, CHUNK), jnp.int32),
        pltpu.VMEM((N_TCH, IDX_CHUNK), jnp.int32),
        pltpu.VMEM((N_TCH, IDX_CHUNK), jnp.int32),
        pltpu.VMEM((FLAT_PER_W, CHUNK), jnp.float32),
        pltpu.VMEM((FLAT_PER_W, CHUNK), jnp.float32),
        pltpu.SemaphoreType.DMA,
        pltpu.SemaphoreType.DMA,
    ],
    compiler_params=pltpu.CompilerParams(
        use_tc_tiling_on_sc=False, needs_layout_passes=False),
)
def _cc_embed(x_hbm, t0_hbm, t1_hbm, h0_hbm, h1_hbm, out_hbm,
              xv, xh, rr0, rr1, f0, f1, g0, g1, sem0, sem1):
    wid = lax.axis_index("s") * NC + lax.axis_index("c")
    lane = lax.iota(jnp.int32, L)
    col = lane & (N_CHUNKS - 1)

    pltpu.sync_copy(x_hbm.at[pl.ds(wid * B_PER_W, B_PER_W)], xv)
    for j in range(N_XCH):
        for m in range(IDX_CHUNK // L):
            v = xv[pl.ds(j * IDX_CHUNK + m * L, L)]
            xh[j, pl.ds(m * L, L)] = lax.shift_right_logical(v, 2)
    pl.delay(128)
    cps = []
    for j in range(N_XCH):
        cps.append(pltpu.async_copy(
            h0_hbm.at[xh.at[j]], rr0.at[pl.ds(j * IDX_CHUNK, IDX_CHUNK)], sem0))
        cps.append(pltpu.async_copy(
            h1_hbm.at[xh.at[j]], rr1.at[pl.ds(j * IDX_CHUNK, IDX_CHUNK)], sem1))
    for c in cps:
        c.wait()

    for j in range(N_TCH):
        for k in range(IDX_CHUNK // L):
            base = j * IDX_CHUNK + k * L
            bvec = lax.shift_right_logical(base + lane, 2)
            xq = plsc.load_gather(xv, [bvec])
            ci = (xq & (N_CHUNKS - 1)) * N_CHUNKS + col
            v0 = plsc.load_gather(rr0, [bvec, ci])
            v1 = plsc.load_gather(rr1, [bvec, ci])
            f0[j, pl.ds(k * L, L)] = v0 * N_CHUNKS + col
            f1[j, pl.ds(k * L, L)] = v1 * N_CHUNKS + col

    pl.delay(128)
    for b in range(0, N_TCH, 4):
        cps = []
        for j in range(b, b + 4):
            cps.append(pltpu.async_copy(
                t0_hbm.at[f0.at[j]], g0.at[pl.ds(j * IDX_CHUNK, IDX_CHUNK)], sem0))
            cps.append(pltpu.async_copy(
                t1_hbm.at[f1.at[j]], g1.at[pl.ds(j * IDX_CHUNK, IDX_CHUNK)], sem1))
        for c in cps:
            c.wait()

    @pl.loop(0, FLAT_PER_W, step=8)
    def _add(i):
        for u in range(8):
            row = i + u
            g0[row, :] = g0[row, :] + g1[row, :]

    pltpu.sync_copy(g0, out_hbm.at[pl.ds(wid * FLAT_PER_W, FLAT_PER_W)])


def kernel(x, table0, table1, h0, h1):
    t0f = table0.reshape(ROWS * N_CHUNKS, CHUNK)
    t1f = table1.reshape(ROWS * N_CHUNKS, CHUNK)
    h0r = h0.reshape(VOCAB // N_CHUNKS, N_CHUNKS * N_CHUNKS)
    h1r = h1.reshape(VOCAB // N_CHUNKS, N_CHUNKS * N_CHUNKS)
    out = _cc_embed(x, t0f, t1f, h0r, h1r)
    return out.reshape(BATCH, N_CHUNKS * CHUNK)
